# Initial kernel scaffold; baseline (speedup 1.0000x reference)
import jax
import jax.numpy as jnp
from jax.experimental import pallas as pl


def kernel(x, adj_mask, Wq, Wk, Wv, Wo, ln1_g, ln1_b, ln2_g, ln2_b, W1, b1, W2, b2):
    raise NotImplementedError("write your pallas kernel here")


# speedup vs baseline: 1.8019x; 1.8019x over previous
import functools

import jax
import jax.numpy as jnp
from jax.experimental import pallas as pl

_NUM_HEADS = 16
_EPS = 1e-5


def _ln(x, g, b):
    mu = jnp.mean(x, axis=-1, keepdims=True)
    xc = x - mu
    var = jnp.mean(xc * xc, axis=-1, keepdims=True)
    return xc * jax.lax.rsqrt(var + _EPS) * g + b


def _ln_qkv_body(x_ref, g_ref, b_ref, wq_ref, wk_ref, wv_ref,
                 q_ref, k_ref, v_ref):
    h = _ln(x_ref[...], g_ref[...], b_ref[...]).astype(jnp.bfloat16)
    q_ref[...] = jnp.dot(h, wq_ref[...],
                         preferred_element_type=jnp.float32).astype(jnp.bfloat16)
    k_ref[...] = jnp.dot(h, wk_ref[...],
                         preferred_element_type=jnp.float32).astype(jnp.bfloat16)
    v_ref[...] = jnp.dot(h, wv_ref[...],
                         preferred_element_type=jnp.float32).astype(jnp.bfloat16)


def _attn_body(q_ref, k_ref, v_ref, m_ref, o_ref, *, dh, scale):
    num_heads = q_ref.shape[-1] // dh
    mask = m_ref[0]
    outs = []
    for h in range(num_heads):
        sl = slice(h * dh, (h + 1) * dh)
        qh = q_ref[0, :, sl] * jnp.bfloat16(scale)
        kh = k_ref[0, :, sl]
        vh = v_ref[0, :, sl]
        s = jax.lax.dot_general(qh, kh, (((1,), (1,)), ((), ())),
                                preferred_element_type=jnp.float32)
        s = s + mask
        m = jnp.max(s, axis=-1, keepdims=True)
        p = jnp.exp(s - m)
        l = jnp.sum(p, axis=-1, keepdims=True)
        oh = jnp.dot(p.astype(jnp.bfloat16), vh,
                     preferred_element_type=jnp.float32)
        outs.append(oh / l)
    o_ref[0] = jnp.concatenate(outs, axis=-1).astype(jnp.bfloat16)


def _out_ffn_body(x_ref, o_ref, wo_ref, g2_ref, b2g_ref, w1_ref, b1_ref,
                  w2_ref, b2_ref, y_ref):
    xo = x_ref[...] + jnp.dot(o_ref[...], wo_ref[...],
                              preferred_element_type=jnp.float32)
    h2 = _ln(xo, g2_ref[...], b2g_ref[...]).astype(jnp.bfloat16)
    a = jnp.dot(h2, w1_ref[...], preferred_element_type=jnp.float32)
    g = jax.nn.gelu(a + b1_ref[...]).astype(jnp.bfloat16)
    m = jnp.dot(g, w2_ref[...], preferred_element_type=jnp.float32)
    y_ref[...] = xo + m + b2_ref[...]


def kernel(x, adj_mask, Wq, Wk, Wv, Wo, ln1_g, ln1_b, ln2_g, ln2_b,
           W1, b1, W2, b2):
    B, S, D = x.shape
    H = _NUM_HEADS
    DH = D // H
    FF = W1.shape[1]

    bq1 = min(512, S)
    bq2 = min(256, S)
    bq3 = min(256, S)

    wq = Wq.astype(jnp.bfloat16)
    wk = Wk.astype(jnp.bfloat16)
    wv = Wv.astype(jnp.bfloat16)
    wo = Wo.astype(jnp.bfloat16)
    w1 = W1.astype(jnp.bfloat16)
    w2 = W2.astype(jnp.bfloat16)
    x2 = x.reshape(B * S, D)
    g1 = ln1_g.reshape(1, D)
    bb1 = ln1_b.reshape(1, D)
    g2 = ln2_g.reshape(1, D)
    bb2 = ln2_b.reshape(1, D)
    bias1 = b1.reshape(1, FF)
    bias2 = b2.reshape(1, D)
    mask = adj_mask.reshape(B, S, S)

    rows = pl.BlockSpec((bq1, D), lambda i: (i, 0))
    full = pl.BlockSpec((D, D), lambda i: (0, 0))
    vec = pl.BlockSpec((1, D), lambda i: (0, 0))
    qkv_shape = jax.ShapeDtypeStruct((B * S, D), jnp.bfloat16)
    q2, k2, v2 = pl.pallas_call(
        _ln_qkv_body,
        grid=(B * S // bq1,),
        in_specs=[rows, vec, vec, full, full, full],
        out_specs=[rows, rows, rows],
        out_shape=[qkv_shape, qkv_shape, qkv_shape],
    )(x2, g1, bb1, wq, wk, wv)

    q3 = q2.reshape(B, S, D)
    k3 = k2.reshape(B, S, D)
    v3 = v2.reshape(B, S, D)

    scale = 1.0 / (DH ** 0.5)
    qspec = pl.BlockSpec((1, bq2, D), lambda b, i: (b, i, 0))
    kvspec = pl.BlockSpec((1, S, D), lambda b, i: (b, 0, 0))
    mspec = pl.BlockSpec((1, bq2, S), lambda b, i: (b, i, 0))
    o3 = pl.pallas_call(
        functools.partial(_attn_body, dh=DH, scale=scale),
        grid=(B, S // bq2),
        in_specs=[qspec, kvspec, kvspec, mspec],
        out_specs=qspec,
        out_shape=jax.ShapeDtypeStruct((B, S, D), jnp.bfloat16),
    )(q3, k3, v3, mask)

    rows3 = pl.BlockSpec((bq3, D), lambda i: (i, 0))
    y = pl.pallas_call(
        _out_ffn_body,
        grid=(B * S // bq3,),
        in_specs=[rows3, rows3,
                  pl.BlockSpec((D, D), lambda i: (0, 0)),
                  pl.BlockSpec((1, D), lambda i: (0, 0)),
                  pl.BlockSpec((1, D), lambda i: (0, 0)),
                  pl.BlockSpec((D, FF), lambda i: (0, 0)),
                  pl.BlockSpec((1, FF), lambda i: (0, 0)),
                  pl.BlockSpec((FF, D), lambda i: (0, 0)),
                  pl.BlockSpec((1, D), lambda i: (0, 0))],
        out_specs=rows3,
        out_shape=jax.ShapeDtypeStruct((B * S, D), jnp.float32),
    )(x2, o3.reshape(B * S, D), wo, g2, bb2, w1, bias1, w2, bias2)

    return y.reshape(B, S, D)


# speedup vs baseline: 2.0641x; 1.1455x over previous
import functools

import jax
import jax.numpy as jnp
from jax.experimental import pallas as pl

_NUM_HEADS = 16
_EPS = 1e-5
_CLAMP = 30.0


def _ln(x, g, b):
    mu = jnp.mean(x, axis=-1, keepdims=True)
    xc = x - mu
    var = jnp.mean(xc * xc, axis=-1, keepdims=True)
    return xc * jax.lax.rsqrt(var + _EPS) * g + b


def _ln_qkv_body(x_ref, g_ref, b_ref, w_ref, vb_ref, q_ref, k_ref, v_ref, *, d):
    h = _ln(x_ref[...], g_ref[...], b_ref[...]).astype(jnp.bfloat16)
    r = jnp.dot(h, w_ref[...], preferred_element_type=jnp.float32)
    q_ref[...] = r[:, :d].astype(jnp.bfloat16)
    k_ref[...] = r[:, d:2 * d].astype(jnp.bfloat16)
    v_ref[...] = (r[:, 2 * d:] + vb_ref[...]).astype(jnp.bfloat16)


def _attn_body(q_ref, k_ref, v_ref, m_ref, o_ref, *, dh, scale):
    num_heads = q_ref.shape[-1] // dh
    mask = m_ref[0]
    outs = []
    for h in range(num_heads):
        qh = q_ref[0, :, h * dh:(h + 1) * dh] * jnp.bfloat16(scale)
        kh = k_ref[0, :, h * dh:(h + 1) * dh]
        vh = v_ref[0, :, h * 2 * dh:(h + 1) * 2 * dh]
        s = jax.lax.dot_general(qh, kh, (((1,), (1,)), ((), ())),
                                preferred_element_type=jnp.float32)
        sb = jnp.minimum(s, _CLAMP).astype(jnp.bfloat16)
        p = jnp.exp(sb) * mask
        ol = jnp.dot(p, vh, preferred_element_type=jnp.float32)
        l = jnp.maximum(ol[:, dh:dh + 1], 1e-30)
        outs.append(ol[:, :dh] * (1.0 / l))
    o_ref[0] = jnp.concatenate(outs, axis=-1).astype(jnp.bfloat16)


def _out_ffn_body(x_ref, o_ref, wo_ref, g2_ref, b2g_ref, w1_ref, b1_ref,
                  w2_ref, b2_ref, y_ref):
    xo = x_ref[...] + jnp.dot(o_ref[...], wo_ref[...],
                              preferred_element_type=jnp.float32)
    h2 = _ln(xo, g2_ref[...], b2g_ref[...]).astype(jnp.bfloat16)
    a = jnp.dot(h2, w1_ref[...], preferred_element_type=jnp.float32)
    g = jax.nn.gelu(a + b1_ref[...]).astype(jnp.bfloat16)
    m = jnp.dot(g, w2_ref[...], preferred_element_type=jnp.float32)
    y_ref[...] = xo + m + b2_ref[...]


def kernel(x, adj_mask, Wq, Wk, Wv, Wo, ln1_g, ln1_b, ln2_g, ln2_b,
           W1, b1, W2, b2):
    B, S, D = x.shape
    H = _NUM_HEADS
    DH = D // H
    FF = W1.shape[1]

    bq1 = min(512, S)
    bq2 = min(256, S)
    bq3 = min(256, S)

    wv_aug = jnp.zeros((D, 2 * D), jnp.float32)
    wv_aug = wv_aug.at[:, (jnp.arange(D) // DH) * 2 * DH + (jnp.arange(D) % DH)].set(Wv)
    v_bias = jnp.zeros((1, 2 * D), jnp.float32)
    v_bias = v_bias.at[0, jnp.arange(H) * 2 * DH + DH].set(1.0)
    w_all = jnp.concatenate([Wq, Wk, wv_aug], axis=1).astype(jnp.bfloat16)

    x2 = x.reshape(B * S, D)
    g1 = ln1_g.reshape(1, D)
    bb1 = ln1_b.reshape(1, D)
    g2 = ln2_g.reshape(1, D)
    bb2 = ln2_b.reshape(1, D)
    bias1 = b1.reshape(1, FF)
    bias2 = b2.reshape(1, D)
    mask01 = (adj_mask.reshape(B, S, S) == 0.0).astype(jnp.bfloat16)

    rows = pl.BlockSpec((bq1, D), lambda i: (i, 0))
    rows2 = pl.BlockSpec((bq1, 2 * D), lambda i: (i, 0))
    q2, k2, v2 = pl.pallas_call(
        functools.partial(_ln_qkv_body, d=D),
        grid=(B * S // bq1,),
        in_specs=[rows,
                  pl.BlockSpec((1, D), lambda i: (0, 0)),
                  pl.BlockSpec((1, D), lambda i: (0, 0)),
                  pl.BlockSpec((D, 4 * D), lambda i: (0, 0)),
                  pl.BlockSpec((1, 2 * D), lambda i: (0, 0))],
        out_specs=[rows, rows, rows2],
        out_shape=[jax.ShapeDtypeStruct((B * S, D), jnp.bfloat16),
                   jax.ShapeDtypeStruct((B * S, D), jnp.bfloat16),
                   jax.ShapeDtypeStruct((B * S, 2 * D), jnp.bfloat16)],
    )(x2, g1, bb1, w_all, v_bias)

    q3 = q2.reshape(B, S, D)
    k3 = k2.reshape(B, S, D)
    v3 = v2.reshape(B, S, 2 * D)

    scale = 1.0 / (DH ** 0.5)
    qspec = pl.BlockSpec((1, bq2, D), lambda b, i: (b, i, 0))
    kspec = pl.BlockSpec((1, S, D), lambda b, i: (b, 0, 0))
    vspec = pl.BlockSpec((1, S, 2 * D), lambda b, i: (b, 0, 0))
    mspec = pl.BlockSpec((1, bq2, S), lambda b, i: (b, i, 0))
    o3 = pl.pallas_call(
        functools.partial(_attn_body, dh=DH, scale=scale),
        grid=(B, S // bq2),
        in_specs=[qspec, kspec, vspec, mspec],
        out_specs=qspec,
        out_shape=jax.ShapeDtypeStruct((B, S, D), jnp.bfloat16),
    )(q3, k3, v3, mask01)

    rows3 = pl.BlockSpec((bq3, D), lambda i: (i, 0))
    y = pl.pallas_call(
        _out_ffn_body,
        grid=(B * S // bq3,),
        in_specs=[rows3, rows3,
                  pl.BlockSpec((D, D), lambda i: (0, 0)),
                  pl.BlockSpec((1, D), lambda i: (0, 0)),
                  pl.BlockSpec((1, D), lambda i: (0, 0)),
                  pl.BlockSpec((D, FF), lambda i: (0, 0)),
                  pl.BlockSpec((1, FF), lambda i: (0, 0)),
                  pl.BlockSpec((FF, D), lambda i: (0, 0)),
                  pl.BlockSpec((1, D), lambda i: (0, 0))],
        out_specs=rows3,
        out_shape=jax.ShapeDtypeStruct((B * S, D), jnp.float32),
    )(x2, o3.reshape(B * S, D), Wo.astype(jnp.bfloat16), g2, bb2,
      W1.astype(jnp.bfloat16), bias1, W2.astype(jnp.bfloat16), bias2)

    return y.reshape(B, S, D)


# speedup vs baseline: 2.1243x; 1.0292x over previous
import functools

import jax
import jax.numpy as jnp
from jax.experimental import pallas as pl

_NUM_HEADS = 16
_EPS = 1e-5
_CLAMP = 30.0


def _ln(x, g, b):
    mu = jnp.mean(x, axis=-1, keepdims=True)
    xc = x - mu
    var = jnp.mean(xc * xc, axis=-1, keepdims=True)
    return xc * jax.lax.rsqrt(var + _EPS) * g + b


def _ln_qkv_body(x_ref, g_ref, b_ref, w_ref, vb_ref, q_ref, kt_ref, v_ref, *, d):
    h = _ln(x_ref[...], g_ref[...], b_ref[...]).astype(jnp.bfloat16)
    r = jnp.dot(h, w_ref[...], preferred_element_type=jnp.float32)
    q_ref[...] = r[:, :d].astype(jnp.bfloat16)
    kt_ref[0] = r[:, d:2 * d].astype(jnp.bfloat16).T
    v_ref[...] = (r[:, 2 * d:] + vb_ref[...]).astype(jnp.bfloat16)


def _attn_ffn_body(q_ref, kt_ref, v_ref, m_ref, x_ref, wo_ref, g2_ref,
                   b2g_ref, w1_ref, b1_ref, w2_ref, b2_ref, y_ref,
                   *, dh, scale):
    num_heads = q_ref.shape[-1] // dh
    mask = m_ref[0]
    outs = []
    for h in range(num_heads):
        qh = q_ref[0, :, h * dh:(h + 1) * dh] * jnp.bfloat16(scale)
        kth = kt_ref[0, h * dh:(h + 1) * dh, :]
        vh = v_ref[0, :, h * 2 * dh:(h + 1) * 2 * dh]
        s = jnp.dot(qh, kth, preferred_element_type=jnp.float32)
        sb = jnp.minimum(s, _CLAMP).astype(jnp.bfloat16)
        p = jnp.exp(sb) * mask
        ol = jnp.dot(p, vh, preferred_element_type=jnp.float32)
        l = jnp.maximum(ol[:, dh:dh + 1], 1e-30)
        outs.append(ol[:, :dh] * (1.0 / l))
    o = jnp.concatenate(outs, axis=-1).astype(jnp.bfloat16)
    xo = x_ref[0] + jnp.dot(o, wo_ref[...], preferred_element_type=jnp.float32)
    h2 = _ln(xo, g2_ref[...], b2g_ref[...]).astype(jnp.bfloat16)
    a = jnp.dot(h2, w1_ref[...], preferred_element_type=jnp.float32)
    g = jax.nn.gelu(a + b1_ref[...]).astype(jnp.bfloat16)
    m = jnp.dot(g, w2_ref[...], preferred_element_type=jnp.float32)
    y_ref[0] = xo + m + b2_ref[...]


def kernel(x, adj_mask, Wq, Wk, Wv, Wo, ln1_g, ln1_b, ln2_g, ln2_b,
           W1, b1, W2, b2):
    B, S, D = x.shape
    H = _NUM_HEADS
    DH = D // H
    FF = W1.shape[1]

    bq1 = min(512, S)
    bq2 = min(256, S)

    wv_aug = jnp.zeros((D, 2 * D), jnp.float32)
    wv_aug = wv_aug.at[:, (jnp.arange(D) // DH) * 2 * DH + (jnp.arange(D) % DH)].set(Wv)
    v_bias = jnp.zeros((1, 2 * D), jnp.float32)
    v_bias = v_bias.at[0, jnp.arange(H) * 2 * DH + DH].set(1.0)
    w_all = jnp.concatenate([Wq, Wk, wv_aug], axis=1).astype(jnp.bfloat16)

    x3 = x
    x2 = x.reshape(B * S, D)
    g1 = ln1_g.reshape(1, D)
    bb1 = ln1_b.reshape(1, D)
    g2 = ln2_g.reshape(1, D)
    bb2 = ln2_b.reshape(1, D)
    bias1 = b1.reshape(1, FF)
    bias2 = b2.reshape(1, D)
    mask01 = (adj_mask.reshape(B, S, S) == 0.0).astype(jnp.bfloat16)

    nq1 = S // bq1
    rows = pl.BlockSpec((bq1, D), lambda i: (i, 0))
    rows2 = pl.BlockSpec((bq1, 2 * D), lambda i: (i, 0))
    ktspec = pl.BlockSpec((1, D, bq1), lambda i: (i // nq1, 0, i % nq1))
    q2, kt, v2 = pl.pallas_call(
        functools.partial(_ln_qkv_body, d=D),
        grid=(B * S // bq1,),
        in_specs=[rows,
                  pl.BlockSpec((1, D), lambda i: (0, 0)),
                  pl.BlockSpec((1, D), lambda i: (0, 0)),
                  pl.BlockSpec((D, 4 * D), lambda i: (0, 0)),
                  pl.BlockSpec((1, 2 * D), lambda i: (0, 0))],
        out_specs=[rows, ktspec, rows2],
        out_shape=[jax.ShapeDtypeStruct((B * S, D), jnp.bfloat16),
                   jax.ShapeDtypeStruct((B, D, S), jnp.bfloat16),
                   jax.ShapeDtypeStruct((B * S, 2 * D), jnp.bfloat16)],
    )(x2, g1, bb1, w_all, v_bias)

    q3 = q2.reshape(B, S, D)
    v3 = v2.reshape(B, S, 2 * D)

    scale = 1.0 / (DH ** 0.5)
    qspec = pl.BlockSpec((1, bq2, D), lambda b, i: (b, i, 0))
    ktspec2 = pl.BlockSpec((1, D, S), lambda b, i: (b, 0, 0))
    vspec = pl.BlockSpec((1, S, 2 * D), lambda b, i: (b, 0, 0))
    mspec = pl.BlockSpec((1, bq2, S), lambda b, i: (b, i, 0))
    const = lambda shape: pl.BlockSpec(shape, lambda b, i: tuple(0 for _ in shape))
    y = pl.pallas_call(
        functools.partial(_attn_ffn_body, dh=DH, scale=scale),
        grid=(B, S // bq2),
        in_specs=[qspec, ktspec2, vspec, mspec, qspec,
                  const((D, D)), const((1, D)), const((1, D)),
                  const((D, FF)), const((1, FF)),
                  const((FF, D)), const((1, D))],
        out_specs=qspec,
        out_shape=jax.ShapeDtypeStruct((B, S, D), jnp.float32),
    )(q3, kt, v3, mask01, x3, Wo.astype(jnp.bfloat16), g2, bb2,
      W1.astype(jnp.bfloat16), bias1, W2.astype(jnp.bfloat16), bias2)

    return y


# speedup vs baseline: 2.3355x; 1.0994x over previous
import functools

import jax
import jax.numpy as jnp
from jax.experimental import pallas as pl

_NUM_HEADS = 16
_EPS = 1e-5
_CLAMP = 30.0


def _ln(x, g, b):
    mu = jnp.mean(x, axis=-1, keepdims=True)
    ms = jnp.mean(x * x, axis=-1, keepdims=True)
    var = ms - mu * mu
    return (x - mu) * jax.lax.rsqrt(var + _EPS) * g + b


def _ln_qkv_body(x_ref, g_ref, b_ref, w_ref, vb_ref, q_ref, kt_ref, v_ref, *, d):
    h = _ln(x_ref[...], g_ref[...], b_ref[...]).astype(jnp.bfloat16)
    r = jnp.dot(h, w_ref[...], preferred_element_type=jnp.float32)
    q_ref[...] = r[:, :d].astype(jnp.bfloat16)
    kt_ref[0] = r[:, d:2 * d].astype(jnp.bfloat16).T
    v_ref[...] = (r[:, 2 * d:] + vb_ref[...]).astype(jnp.bfloat16)


def _attn_ffn_body(q_ref, kt_ref, v_ref, m_ref, x_ref, wo_ref, g2_ref,
                   b2g_ref, w1_ref, b1_ref, w2_ref, b2_ref, y_ref,
                   *, dh, scale):
    num_heads = q_ref.shape[-1] // dh
    mask = m_ref[0]
    outs = []
    for h in range(num_heads):
        qh = q_ref[0, :, h * dh:(h + 1) * dh] * jnp.bfloat16(scale)
        kth = kt_ref[0, h * dh:(h + 1) * dh, :]
        vh = v_ref[0, :, h * 2 * dh:(h + 1) * 2 * dh]
        s = jnp.dot(qh, kth, preferred_element_type=jnp.float32)
        sb = jnp.minimum(s + mask, _CLAMP).astype(jnp.bfloat16)
        p = jnp.exp(sb)
        ol = jnp.dot(p, vh, preferred_element_type=jnp.float32)
        l = jnp.maximum(ol[:, dh:dh + 1], 1e-30)
        outs.append(ol[:, :dh] * (1.0 / l))
    o = jnp.concatenate(outs, axis=-1).astype(jnp.bfloat16)
    xo = x_ref[0] + jnp.dot(o, wo_ref[...], preferred_element_type=jnp.float32)
    h2 = _ln(xo, g2_ref[...], b2g_ref[...]).astype(jnp.bfloat16)
    a = jnp.dot(h2, w1_ref[...], preferred_element_type=jnp.float32)
    g = jax.nn.gelu(a + b1_ref[...]).astype(jnp.bfloat16)
    m = jnp.dot(g, w2_ref[...], preferred_element_type=jnp.float32)
    y_ref[0] = xo + m + b2_ref[...]


def kernel(x, adj_mask, Wq, Wk, Wv, Wo, ln1_g, ln1_b, ln2_g, ln2_b,
           W1, b1, W2, b2):
    B, S, D = x.shape
    H = _NUM_HEADS
    DH = D // H
    FF = W1.shape[1]

    bq1 = min(512, S)
    bq2 = min(256, S)

    wv_aug = jnp.concatenate(
        [Wv.reshape(D, H, DH), jnp.zeros((D, H, DH), jnp.float32)],
        axis=-1).reshape(D, 2 * D)
    v_bias = jnp.concatenate(
        [jnp.zeros((1, H, DH), jnp.float32), jnp.ones((1, H, 1), jnp.float32),
         jnp.zeros((1, H, DH - 1), jnp.float32)], axis=-1).reshape(1, 2 * D)
    w_all = jnp.concatenate([Wq, Wk, wv_aug], axis=1).astype(jnp.bfloat16)

    x3 = x
    x2 = x.reshape(B * S, D)
    g1 = ln1_g.reshape(1, D)
    bb1 = ln1_b.reshape(1, D)
    g2 = ln2_g.reshape(1, D)
    bb2 = ln2_b.reshape(1, D)
    bias1 = b1.reshape(1, FF)
    bias2 = b2.reshape(1, D)
    mask_add = adj_mask.reshape(B, S, S)

    nq1 = S // bq1
    rows = pl.BlockSpec((bq1, D), lambda i: (i, 0))
    rows2 = pl.BlockSpec((bq1, 2 * D), lambda i: (i, 0))
    ktspec = pl.BlockSpec((1, D, bq1), lambda i: (i // nq1, 0, i % nq1))
    q2, kt, v2 = pl.pallas_call(
        functools.partial(_ln_qkv_body, d=D),
        grid=(B * S // bq1,),
        in_specs=[rows,
                  pl.BlockSpec((1, D), lambda i: (0, 0)),
                  pl.BlockSpec((1, D), lambda i: (0, 0)),
                  pl.BlockSpec((D, 4 * D), lambda i: (0, 0)),
                  pl.BlockSpec((1, 2 * D), lambda i: (0, 0))],
        out_specs=[rows, ktspec, rows2],
        out_shape=[jax.ShapeDtypeStruct((B * S, D), jnp.bfloat16),
                   jax.ShapeDtypeStruct((B, D, S), jnp.bfloat16),
                   jax.ShapeDtypeStruct((B * S, 2 * D), jnp.bfloat16)],
    )(x2, g1, bb1, w_all, v_bias)

    q3 = q2.reshape(B, S, D)
    v3 = v2.reshape(B, S, 2 * D)

    scale = 1.0 / (DH ** 0.5)
    qspec = pl.BlockSpec((1, bq2, D), lambda b, i: (b, i, 0))
    ktspec2 = pl.BlockSpec((1, D, S), lambda b, i: (b, 0, 0))
    vspec = pl.BlockSpec((1, S, 2 * D), lambda b, i: (b, 0, 0))
    mspec = pl.BlockSpec((1, bq2, S), lambda b, i: (b, i, 0))
    const = lambda shape: pl.BlockSpec(shape, lambda b, i: tuple(0 for _ in shape))
    y = pl.pallas_call(
        functools.partial(_attn_ffn_body, dh=DH, scale=scale),
        grid=(B, S // bq2),
        in_specs=[qspec, ktspec2, vspec, mspec, qspec,
                  const((D, D)), const((1, D)), const((1, D)),
                  const((D, FF)), const((1, FF)),
                  const((FF, D)), const((1, D))],
        out_specs=qspec,
        out_shape=jax.ShapeDtypeStruct((B, S, D), jnp.float32),
    )(q3, kt, v3, mask_add, x3, Wo.astype(jnp.bfloat16), g2, bb2,
      W1.astype(jnp.bfloat16), bias1, W2.astype(jnp.bfloat16), bias2)

    return y


# speedup vs baseline: 2.4481x; 1.0482x over previous
import functools

import jax
import jax.numpy as jnp
from jax.experimental import pallas as pl

_NUM_HEADS = 16
_EPS = 1e-5
_CLAMP = 30.0


def _ln(x, g, b):
    mu = jnp.mean(x, axis=-1, keepdims=True)
    ms = jnp.mean(x * x, axis=-1, keepdims=True)
    var = ms - mu * mu
    return (x - mu) * jax.lax.rsqrt(var + _EPS) * g + b


def _ln_qkv_body(x_ref, g_ref, b_ref, w_ref, vb_ref, q_ref, kt_ref, v_ref, *, d):
    h = _ln(x_ref[...], g_ref[...], b_ref[...]).astype(jnp.bfloat16)
    r = jnp.dot(h, w_ref[...], preferred_element_type=jnp.float32)
    q_ref[...] = r[:, :d].astype(jnp.bfloat16)
    kt_ref[0] = r[:, d:2 * d].astype(jnp.bfloat16).T
    v_ref[...] = (r[:, 2 * d:] + vb_ref[...]).astype(jnp.bfloat16)


def _attn_ffn_body(q_ref, kt_ref, v_ref, m_ref, x_ref, wo_ref, g2_ref,
                   b2g_ref, w1_ref, b1_ref, w2_ref, b2_ref, y_ref,
                   *, dh, scale):
    num_heads = q_ref.shape[-1] // dh
    mask = m_ref[0]
    outs = []
    for h in range(num_heads):
        qh = q_ref[0, :, h * dh:(h + 1) * dh] * jnp.bfloat16(scale)
        kth = kt_ref[0, h * dh:(h + 1) * dh, :]
        vh = v_ref[0, :, h * 2 * dh:(h + 1) * 2 * dh]
        s = jnp.dot(qh, kth, preferred_element_type=jnp.float32)
        sb = jnp.minimum(s + mask, _CLAMP).astype(jnp.bfloat16)
        p = jnp.exp(sb)
        ol = jnp.dot(p, vh, preferred_element_type=jnp.float32)
        l = jnp.maximum(ol[:, dh:dh + 1], 1e-30)
        outs.append(ol[:, :dh] * (1.0 / l))
    o = jnp.concatenate(outs, axis=-1).astype(jnp.bfloat16)
    xo = x_ref[0] + jnp.dot(o, wo_ref[...], preferred_element_type=jnp.float32)
    h2 = _ln(xo, g2_ref[...], b2g_ref[...]).astype(jnp.bfloat16)
    a = jnp.dot(h2, w1_ref[...], preferred_element_type=jnp.float32)
    g = jax.nn.gelu(a + b1_ref[...]).astype(jnp.bfloat16)
    m = jnp.dot(g, w2_ref[...], preferred_element_type=jnp.float32)
    y_ref[0] = xo + m + b2_ref[...]


def kernel(x, adj_mask, Wq, Wk, Wv, Wo, ln1_g, ln1_b, ln2_g, ln2_b,
           W1, b1, W2, b2):
    B, S, D = x.shape
    H = _NUM_HEADS
    DH = D // H
    FF = W1.shape[1]

    bq1 = min(512, S)
    bq2 = min(512, S)

    wv_aug = jnp.concatenate(
        [Wv.reshape(D, H, DH), jnp.zeros((D, H, DH), jnp.float32)],
        axis=-1).reshape(D, 2 * D)
    v_bias = jnp.concatenate(
        [jnp.zeros((1, H, DH), jnp.float32), jnp.ones((1, H, 1), jnp.float32),
         jnp.zeros((1, H, DH - 1), jnp.float32)], axis=-1).reshape(1, 2 * D)
    w_all = jnp.concatenate([Wq, Wk, wv_aug], axis=1).astype(jnp.bfloat16)

    x3 = x
    x2 = x.reshape(B * S, D)
    g1 = ln1_g.reshape(1, D)
    bb1 = ln1_b.reshape(1, D)
    g2 = ln2_g.reshape(1, D)
    bb2 = ln2_b.reshape(1, D)
    bias1 = b1.reshape(1, FF)
    bias2 = b2.reshape(1, D)
    mask_add = adj_mask.reshape(B, S, S)

    nq1 = S // bq1
    rows = pl.BlockSpec((bq1, D), lambda i: (i, 0))
    rows2 = pl.BlockSpec((bq1, 2 * D), lambda i: (i, 0))
    ktspec = pl.BlockSpec((1, D, bq1), lambda i: (i // nq1, 0, i % nq1))
    q2, kt, v2 = pl.pallas_call(
        functools.partial(_ln_qkv_body, d=D),
        grid=(B * S // bq1,),
        in_specs=[rows,
                  pl.BlockSpec((1, D), lambda i: (0, 0)),
                  pl.BlockSpec((1, D), lambda i: (0, 0)),
                  pl.BlockSpec((D, 4 * D), lambda i: (0, 0)),
                  pl.BlockSpec((1, 2 * D), lambda i: (0, 0))],
        out_specs=[rows, ktspec, rows2],
        out_shape=[jax.ShapeDtypeStruct((B * S, D), jnp.bfloat16),
                   jax.ShapeDtypeStruct((B, D, S), jnp.bfloat16),
                   jax.ShapeDtypeStruct((B * S, 2 * D), jnp.bfloat16)],
    )(x2, g1, bb1, w_all, v_bias)

    q3 = q2.reshape(B, S, D)
    v3 = v2.reshape(B, S, 2 * D)

    scale = 1.0 / (DH ** 0.5)
    qspec = pl.BlockSpec((1, bq2, D), lambda b, i: (b, i, 0))
    ktspec2 = pl.BlockSpec((1, D, S), lambda b, i: (b, 0, 0))
    vspec = pl.BlockSpec((1, S, 2 * D), lambda b, i: (b, 0, 0))
    mspec = pl.BlockSpec((1, bq2, S), lambda b, i: (b, i, 0))
    const = lambda shape: pl.BlockSpec(shape, lambda b, i: tuple(0 for _ in shape))
    y = pl.pallas_call(
        functools.partial(_attn_ffn_body, dh=DH, scale=scale),
        grid=(B, S // bq2),
        in_specs=[qspec, ktspec2, vspec, mspec, qspec,
                  const((D, D)), const((1, D)), const((1, D)),
                  const((D, FF)), const((1, FF)),
                  const((FF, D)), const((1, D))],
        out_specs=qspec,
        out_shape=jax.ShapeDtypeStruct((B, S, D), jnp.float32),
    )(q3, kt, v3, mask_add, x3, Wo.astype(jnp.bfloat16), g2, bb2,
      W1.astype(jnp.bfloat16), bias1, W2.astype(jnp.bfloat16), bias2)

    return y
